# baseline (speedup 1.0000x reference)
import functools

import jax
import jax.numpy as jnp
from jax import lax
from jax.experimental import pallas as pl
from jax.experimental.pallas import tpu as pltpu
from jax.experimental.pallas import tpu_sc as plsc

_TVS = 1024
_NW = 32
_NL = 16


def _logits_kern(x_ref, wt_ref, b_ref, out_ref):
    rs = jnp.sum(wt_ref[...] * x_ref[...], axis=1, keepdims=True)
    out_ref[...] = jnp.transpose(rs, (1, 0)) + b_ref[...]


def _sc_top1(logits_hbm, vals_out, idxs_out, buf, stage_v, stage_i, *, v, per):
    nchunk_total = v // _NL
    last_per = nchunk_total - (_NW - 1) * per
    c = lax.axis_index("c")
    s = lax.axis_index("s")
    wid = c * 16 + s
    off = wid * (per * _NL)

    @pl.when(wid < _NW - 1)
    def _copy_main():
        pltpu.sync_copy(logits_hbm.at[pl.ds(off, per * _NL)], buf)

    @pl.when(wid == _NW - 1)
    def _copy_last():
        pltpu.sync_copy(
            logits_hbm.at[pl.ds((_NW - 1) * per * _NL, last_per * _NL)],
            buf.at[pl.ds(0, last_per * _NL)],
        )

    my_n = jnp.where(wid == _NW - 1, last_per, per)
    lanes = lax.iota(jnp.int32, _NL)
    bv0 = jnp.full((_NL,), -jnp.inf, jnp.float32)
    bi0 = jnp.zeros((_NL,), jnp.int32)

    def body(j, carry):
        bv, bi = carry
        val = buf[pl.ds(j * _NL, _NL)]
        idx = off + j * _NL + lanes
        upd = val > bv
        return jnp.where(upd, val, bv), jnp.where(upd, idx, bi)

    bv, bi = lax.fori_loop(0, my_n, body, (bv0, bi0))
    stage_v[...] = bv
    stage_i[...] = bi
    pltpu.sync_copy(stage_v, vals_out.at[wid])
    pltpu.sync_copy(stage_i, idxs_out.at[wid])


def _merge_kern(vals_ref, idxs_ref, out_ref):
    vals = vals_ref[...]
    m = jnp.max(vals)
    out_ref[0] = jnp.min(
        jnp.where(vals == m, idxs_ref[...], jnp.iinfo(jnp.int32).max)
    )


def kernel(x, W, b):
    d, v = W.shape
    tvs = min(_TVS, v)
    nj = pl.cdiv(v, tvs)
    wt = W.T
    x0 = x[0:1]
    b2 = b.reshape(1, v)

    logits = pl.pallas_call(
        _logits_kern,
        grid=(nj,),
        in_specs=[
            pl.BlockSpec((1, d), lambda j: (0, 0)),
            pl.BlockSpec((tvs, d), lambda j: (j, 0)),
            pl.BlockSpec((1, tvs), lambda j: (0, j)),
        ],
        out_specs=pl.BlockSpec((1, tvs), lambda j: (0, j)),
        out_shape=jax.ShapeDtypeStruct((1, v), jnp.float32),
        compiler_params=pltpu.CompilerParams(
            dimension_semantics=("arbitrary",),
        ),
    )(x0, wt, b2)

    nchunk_total = v // _NL
    per = -(-nchunk_total // _NW)
    mesh = plsc.VectorSubcoreMesh(core_axis_name="c", subcore_axis_name="s")
    sc = functools.partial(
        pl.kernel,
        out_type=[
            jax.ShapeDtypeStruct((_NW, _NL), jnp.float32),
            jax.ShapeDtypeStruct((_NW, _NL), jnp.int32),
        ],
        mesh=mesh,
        scratch_types=[
            pltpu.VMEM((per * _NL,), jnp.float32),
            pltpu.VMEM((_NL,), jnp.float32),
            pltpu.VMEM((_NL,), jnp.int32),
        ],
    )(functools.partial(_sc_top1, v=v, per=per))
    vals, idxs = sc(logits.reshape(v))

    out = pl.pallas_call(
        _merge_kern,
        in_specs=[
            pl.BlockSpec(memory_space=pltpu.VMEM),
            pl.BlockSpec(memory_space=pltpu.VMEM),
        ],
        out_specs=pl.BlockSpec(memory_space=pltpu.SMEM),
        out_shape=jax.ShapeDtypeStruct((1,), jnp.int32),
    )(vals, idxs)
    return out


# speedup vs baseline: 1.0923x; 1.0923x over previous
import functools

import jax
import jax.numpy as jnp
from jax.experimental import pallas as pl
from jax.experimental.pallas import tpu as pltpu

_TVS = 1024


def _topk_kern(x_ref, wt_ref, b_ref, out_ref, best, vidx, *, v_total, tvs):
    j = pl.program_id(0)
    nj = pl.num_programs(0)

    rs = jnp.sum(wt_ref[...] * x_ref[...], axis=1, keepdims=True)
    score = jnp.transpose(rs, (1, 0)) + b_ref[...]
    lane = jax.lax.broadcasted_iota(jnp.int32, score.shape, 1)
    gv = j * tvs + lane
    score = jnp.where(gv < v_total, score, -jnp.inf)

    @pl.when(j == 0)
    def _init():
        best[...] = score
        vidx[...] = gv

    @pl.when(j > 0)
    def _update():
        upd = score > best[...]
        best[...] = jnp.where(upd, score, best[...])
        vidx[...] = jnp.where(upd, gv, vidx[...])

    @pl.when(j == nj - 1)
    def _emit():
        b_all = best[...]
        m = jnp.max(b_all)
        out_ref[0] = jnp.min(
            jnp.where(b_all == m, vidx[...], jnp.iinfo(jnp.int32).max)
        )


def kernel(x, W, b):
    d, v = W.shape
    tvs = min(_TVS, v)
    nj = pl.cdiv(v, tvs)
    wt = W.T
    x0 = x[0:1]
    b2 = b.reshape(1, v)
    out = pl.pallas_call(
        functools.partial(_topk_kern, v_total=v, tvs=tvs),
        grid=(nj,),
        in_specs=[
            pl.BlockSpec((1, d), lambda j: (0, 0)),
            pl.BlockSpec((tvs, d), lambda j: (j, 0)),
            pl.BlockSpec((1, tvs), lambda j: (0, j)),
        ],
        out_specs=pl.BlockSpec(memory_space=pltpu.SMEM),
        out_shape=jax.ShapeDtypeStruct((1,), jnp.int32),
        scratch_shapes=[
            pltpu.VMEM((1, tvs), jnp.float32),
            pltpu.VMEM((1, tvs), jnp.int32),
        ],
        compiler_params=pltpu.CompilerParams(
            dimension_semantics=("arbitrary",),
        ),
    )(x0, wt, b2)
    return out
